# baseline (speedup 1.0000x reference)
import functools

import jax
import jax.numpy as jnp
from jax import lax
from jax.experimental import pallas as pl
from jax.experimental.pallas import tpu as pltpu
from jax.experimental.pallas import tpu_sc as plsc

T = 2048
D = 768
F = 768
E = 64
K = 2
BLOCK = 128
NB = (T * K) // BLOCK + E
MAX_ROWS = NB * BLOCK
CH = 128
NCH = T // CH

NW = 32
TPW = T // NW
L = 16


def _router_body(x_ref, wr_ref, pos0_ref, pos1_ref, w0_ref, w1_ref, map_ref):
    x = x_ref[...]
    logits = jnp.dot(x, wr_ref[...], preferred_element_type=jnp.float32)

    eio = lax.broadcasted_iota(jnp.int32, (T, E), 1)
    m1 = jnp.max(logits, axis=1, keepdims=True)
    e1 = jnp.min(jnp.where(logits >= m1, eio, E), axis=1)
    masked = jnp.where(eio == e1[:, None], -jnp.inf, logits)
    m2 = jnp.max(masked, axis=1, keepdims=True)
    e2 = jnp.min(jnp.where(masked >= m2, eio, E), axis=1)

    g1 = 1.0 / (1.0 + jnp.exp(m2[:, 0] - m1[:, 0]))
    w0_ref[...] = jnp.broadcast_to(g1[:, None], (T, L))
    w1_ref[...] = jnp.broadcast_to((1.0 - g1)[:, None], (T, L))

    oh1 = (eio == e1[:, None]).astype(jnp.float32)
    oh2 = (eio == e2[:, None]).astype(jnp.float32)
    m = oh1 + oh2

    r = lax.broadcasted_iota(jnp.int32, (CH, CH), 0)
    c = lax.broadcasted_iota(jnp.int32, (CH, CH), 1)
    lt = (r > c).astype(jnp.float32)
    r16 = lax.broadcasted_iota(jnp.int32, (NCH, NCH), 0)
    c16 = lax.broadcasted_iota(jnp.int32, (NCH, NCH), 1)
    lt16 = (r16 > c16).astype(jnp.float32)

    cs, tots = [], []
    for i in range(NCH):
        mc = m[i * CH:(i + 1) * CH]
        cs.append(jnp.dot(lt, mc, preferred_element_type=jnp.float32))
        tots.append(jnp.sum(mc, axis=0, keepdims=True))
    tots = jnp.concatenate(tots, axis=0)
    pref = jnp.dot(lt16, tots, preferred_element_type=jnp.float32)
    cum = jnp.concatenate([cs[i] + pref[i][None, :] for i in range(NCH)], axis=0)

    hist = jnp.sum(tots, axis=0)
    ph = jnp.floor((hist + (BLOCK - 1)) * (1.0 / BLOCK)) * BLOCK
    r64 = lax.broadcasted_iota(jnp.int32, (E, E), 0)
    c64 = lax.broadcasted_iota(jnp.int32, (E, E), 1)
    lt64 = (r64 > c64).astype(jnp.float32)
    off = jnp.dot(lt64, ph[:, None], preferred_element_type=jnp.float32)[:, 0]
    cum_end = off + ph

    rank1 = jnp.sum(oh1 * cum, axis=1)
    rank2 = jnp.sum(oh2 * cum, axis=1)
    pos0_ref[...] = (jnp.sum(oh1 * off[None, :], axis=1) + rank1).astype(jnp.int32)
    pos1_ref[...] = (jnp.sum(oh2 * off[None, :], axis=1) + rank2).astype(jnp.int32)

    bio = (lax.broadcasted_iota(jnp.int32, (NB, E), 0) * BLOCK).astype(jnp.float32)
    map_ref[...] = jnp.sum(
        (bio >= cum_end[None, :]).astype(jnp.float32), axis=1).astype(jnp.int32)


def _router(x, w_router):
    return pl.pallas_call(
        _router_body,
        out_shape=(
            jax.ShapeDtypeStruct((T,), jnp.int32),
            jax.ShapeDtypeStruct((T,), jnp.int32),
            jax.ShapeDtypeStruct((T, L), jnp.float32),
            jax.ShapeDtypeStruct((T, L), jnp.float32),
            jax.ShapeDtypeStruct((NB,), jnp.int32),
        ),
    )(x, w_router)


def _dispatch_body(x_hbm, pos0_hbm, pos1_hbm, xs_hbm, idx0_v, idx1_v, rows_v, sem):
    wid = lax.axis_index("s") * 2 + lax.axis_index("c")
    base = wid * TPW
    a0 = pltpu.async_copy(pos0_hbm.at[pl.ds(base, TPW)], idx0_v, sem)
    a1 = pltpu.async_copy(pos1_hbm.at[pl.ds(base, TPW)], idx1_v, sem)
    a2 = pltpu.async_copy(x_hbm.at[pl.ds(base, TPW)], rows_v, sem)
    a0.wait()
    a1.wait()
    a2.wait()
    c0 = pltpu.async_copy(rows_v, xs_hbm.at[idx0_v], sem)
    c1 = pltpu.async_copy(rows_v, xs_hbm.at[idx1_v], sem)
    c0.wait()
    c1.wait()


@functools.cache
def _make_dispatch():
    return pl.kernel(
        _dispatch_body,
        mesh=plsc.VectorSubcoreMesh(core_axis_name="c", subcore_axis_name="s"),
        out_type=jax.ShapeDtypeStruct((MAX_ROWS + BLOCK, D), jnp.float32),
        scratch_types=[
            pltpu.VMEM((TPW,), jnp.int32),
            pltpu.VMEM((TPW,), jnp.int32),
            pltpu.VMEM((TPW, D), jnp.float32),
            pltpu.SemaphoreType.DMA,
        ],
    )


def _dispatch(x, pos0, pos1):
    return _make_dispatch()(x, pos0, pos1)


def _gemm_body(map_ref, xs_ref, g_ref, u_ref, d_ref, ys_ref):
    b = pl.program_id(0)

    @pl.when(map_ref[b] < E)
    def _():
        xb = xs_ref[...].astype(jnp.bfloat16)
        g = jnp.dot(xb, g_ref[0].astype(jnp.bfloat16),
                    preferred_element_type=jnp.float32)
        u = jnp.dot(xb, u_ref[0].astype(jnp.bfloat16),
                    preferred_element_type=jnp.float32)
        h = (g / (1.0 + jnp.exp(-g))) * u
        ys_ref[...] = jnp.dot(h.astype(jnp.bfloat16), d_ref[0].astype(jnp.bfloat16),
                              preferred_element_type=jnp.float32)


def _grouped_gemm(block_map, xs, gate_w, up_w, down_w):
    grid_spec = pltpu.PrefetchScalarGridSpec(
        num_scalar_prefetch=1,
        grid=(NB,),
        in_specs=[
            pl.BlockSpec((BLOCK, D), lambda b, m: (jnp.where(m[b] < E, b, NB), 0)),
            pl.BlockSpec((1, D, F), lambda b, m: (jnp.minimum(m[b], E - 1), 0, 0)),
            pl.BlockSpec((1, D, F), lambda b, m: (jnp.minimum(m[b], E - 1), 0, 0)),
            pl.BlockSpec((1, F, D), lambda b, m: (jnp.minimum(m[b], E - 1), 0, 0)),
        ],
        out_specs=pl.BlockSpec((BLOCK, D), lambda b, m: (jnp.where(m[b] < E, b, NB), 0)),
    )
    return pl.pallas_call(
        _gemm_body,
        grid_spec=grid_spec,
        out_shape=jax.ShapeDtypeStruct((MAX_ROWS + BLOCK, D), jnp.float32),
    )(block_map, xs, gate_w, up_w, down_w)


def _combine_body(ys_hbm, pos0_hbm, pos1_hbm, w0_hbm, w1_hbm, out_hbm,
                  idx0_v, idx1_v, w0_v, w1_v, buf0_v, buf1_v, sem):
    wid = lax.axis_index("s") * 2 + lax.axis_index("c")
    base = wid * TPW
    pltpu.sync_copy(pos0_hbm.at[pl.ds(base, TPW)], idx0_v)
    pltpu.sync_copy(pos1_hbm.at[pl.ds(base, TPW)], idx1_v)
    pltpu.sync_copy(w0_hbm.at[pl.ds(base, TPW)], w0_v)
    pltpu.sync_copy(w1_hbm.at[pl.ds(base, TPW)], w1_v)
    H = TPW // 2
    c0a = pltpu.async_copy(ys_hbm.at[idx0_v.at[pl.ds(0, H)]], buf0_v.at[pl.ds(0, H)], sem)
    c1a = pltpu.async_copy(ys_hbm.at[idx1_v.at[pl.ds(0, H)]], buf1_v.at[pl.ds(0, H)], sem)
    c0b = pltpu.async_copy(ys_hbm.at[idx0_v.at[pl.ds(H, H)]], buf0_v.at[pl.ds(H, H)], sem)
    c1b = pltpu.async_copy(ys_hbm.at[idx1_v.at[pl.ds(H, H)]], buf1_v.at[pl.ds(H, H)], sem)

    def row(i, carry):
        a0 = w0_v[i, :]
        a1 = w1_v[i, :]
        for ch in range(D // L):
            sl = pl.ds(ch * L, L)
            buf0_v[i, sl] = a0 * buf0_v[i, sl] + a1 * buf1_v[i, sl]
        return carry

    c0a.wait()
    c1a.wait()
    lax.fori_loop(0, H, row, 0)
    wout = pltpu.async_copy(buf0_v.at[pl.ds(0, H)], out_hbm.at[pl.ds(base, H)], sem)
    c0b.wait()
    c1b.wait()
    lax.fori_loop(H, TPW, row, 0)
    wout.wait()
    pltpu.sync_copy(buf0_v.at[pl.ds(H, H)], out_hbm.at[pl.ds(base + H, H)])


@functools.cache
def _make_combine():
    return pl.kernel(
        _combine_body,
        mesh=plsc.VectorSubcoreMesh(core_axis_name="c", subcore_axis_name="s"),
        out_type=jax.ShapeDtypeStruct((T, D), jnp.float32),
        scratch_types=[
            pltpu.VMEM((TPW,), jnp.int32),
            pltpu.VMEM((TPW,), jnp.int32),
            pltpu.VMEM((TPW, L), jnp.float32),
            pltpu.VMEM((TPW, L), jnp.float32),
            pltpu.VMEM((TPW, D), jnp.float32),
            pltpu.VMEM((TPW, D), jnp.float32),
            pltpu.SemaphoreType.DMA,
        ],
    )


def _combine(ys, pos0, pos1, w0, w1):
    return _make_combine()(ys, pos0, pos1, w0, w1)


def kernel(x, W_router, gate_w, up_w, down_w):
    pos0, pos1, w0, w1, block_map = _router(x, W_router)
    xs = _dispatch(x, pos0, pos1)
    ys = _grouped_gemm(block_map, xs, gate_w, up_w, down_w)
    return _combine(ys, pos0, pos1, w0, w1)


# speedup vs baseline: 1.0063x; 1.0063x over previous
import functools

import jax
import jax.numpy as jnp
from jax import lax
from jax.experimental import pallas as pl
from jax.experimental.pallas import tpu as pltpu
from jax.experimental.pallas import tpu_sc as plsc

T = 2048
D = 768
F = 768
E = 64
K = 2
BLOCK = 128
NB = (T * K) // BLOCK + E
MAX_ROWS = NB * BLOCK
CH = 128
NCH = T // CH

NW = 32
TPW = T // NW
L = 16


def _router_body(x_ref, wr_ref, pos0_ref, pos1_ref, w0_ref, w1_ref, map_ref):
    x = x_ref[...]
    logits = jnp.dot(x, wr_ref[...], preferred_element_type=jnp.float32)

    eio = lax.broadcasted_iota(jnp.int32, (T, E), 1)
    m1 = jnp.max(logits, axis=1, keepdims=True)
    e1 = jnp.min(jnp.where(logits >= m1, eio, E), axis=1)
    masked = jnp.where(eio == e1[:, None], -jnp.inf, logits)
    m2 = jnp.max(masked, axis=1, keepdims=True)
    e2 = jnp.min(jnp.where(masked >= m2, eio, E), axis=1)

    g1 = 1.0 / (1.0 + jnp.exp(m2[:, 0] - m1[:, 0]))
    w0_ref[...] = jnp.broadcast_to(g1[:, None], (T, L))
    w1_ref[...] = jnp.broadcast_to((1.0 - g1)[:, None], (T, L))

    oh1 = (eio == e1[:, None]).astype(jnp.float32)
    oh2 = (eio == e2[:, None]).astype(jnp.float32)
    m = oh1 + oh2

    r = lax.broadcasted_iota(jnp.int32, (CH, CH), 0)
    c = lax.broadcasted_iota(jnp.int32, (CH, CH), 1)
    lt = (r > c).astype(jnp.float32)
    r16 = lax.broadcasted_iota(jnp.int32, (NCH, NCH), 0)
    c16 = lax.broadcasted_iota(jnp.int32, (NCH, NCH), 1)
    lt16 = (r16 > c16).astype(jnp.float32)

    cs, tots = [], []
    for i in range(NCH):
        mc = m[i * CH:(i + 1) * CH]
        cs.append(jnp.dot(lt, mc, preferred_element_type=jnp.float32))
        tots.append(jnp.sum(mc, axis=0, keepdims=True))
    tots = jnp.concatenate(tots, axis=0)
    pref = jnp.dot(lt16, tots, preferred_element_type=jnp.float32)
    cum = jnp.concatenate([cs[i] + pref[i][None, :] for i in range(NCH)], axis=0)

    hist = jnp.sum(tots, axis=0)
    ph = jnp.floor((hist + (BLOCK - 1)) * (1.0 / BLOCK)) * BLOCK
    r64 = lax.broadcasted_iota(jnp.int32, (E, E), 0)
    c64 = lax.broadcasted_iota(jnp.int32, (E, E), 1)
    lt64 = (r64 > c64).astype(jnp.float32)
    off = jnp.dot(lt64, ph[:, None], preferred_element_type=jnp.float32)[:, 0]
    cum_end = off + ph

    rank1 = jnp.sum(oh1 * cum, axis=1)
    rank2 = jnp.sum(oh2 * cum, axis=1)
    pos0_ref[...] = (jnp.sum(oh1 * off[None, :], axis=1) + rank1).astype(jnp.int32)
    pos1_ref[...] = (jnp.sum(oh2 * off[None, :], axis=1) + rank2).astype(jnp.int32)

    bio = (lax.broadcasted_iota(jnp.int32, (NB, E), 0) * BLOCK).astype(jnp.float32)
    map_ref[...] = jnp.sum(
        (bio >= cum_end[None, :]).astype(jnp.float32), axis=1).astype(jnp.int32)


def _router(x, w_router):
    return pl.pallas_call(
        _router_body,
        out_shape=(
            jax.ShapeDtypeStruct((T,), jnp.int32),
            jax.ShapeDtypeStruct((T,), jnp.int32),
            jax.ShapeDtypeStruct((T, L), jnp.float32),
            jax.ShapeDtypeStruct((T, L), jnp.float32),
            jax.ShapeDtypeStruct((NB,), jnp.int32),
        ),
    )(x, w_router)


def _dispatch_body(x_hbm, pos0_hbm, pos1_hbm, xs_hbm, idx0_v, idx1_v, rows_v, sem):
    wid = lax.axis_index("s") * 2 + lax.axis_index("c")
    base = wid * TPW
    a0 = pltpu.async_copy(pos0_hbm.at[pl.ds(base, TPW)], idx0_v, sem)
    a1 = pltpu.async_copy(pos1_hbm.at[pl.ds(base, TPW)], idx1_v, sem)
    a2 = pltpu.async_copy(x_hbm.at[pl.ds(base, TPW)], rows_v, sem)
    a0.wait()
    a1.wait()
    a2.wait()
    c0 = pltpu.async_copy(rows_v, xs_hbm.at[idx0_v], sem)
    c1 = pltpu.async_copy(rows_v, xs_hbm.at[idx1_v], sem)
    c0.wait()
    c1.wait()


@functools.cache
def _make_dispatch():
    return pl.kernel(
        _dispatch_body,
        mesh=plsc.VectorSubcoreMesh(core_axis_name="c", subcore_axis_name="s"),
        out_type=jax.ShapeDtypeStruct((MAX_ROWS + BLOCK, D), jnp.float32),
        scratch_types=[
            pltpu.VMEM((TPW,), jnp.int32),
            pltpu.VMEM((TPW,), jnp.int32),
            pltpu.VMEM((TPW, D), jnp.float32),
            pltpu.SemaphoreType.DMA,
        ],
    )


def _dispatch(x, pos0, pos1):
    return _make_dispatch()(x, pos0, pos1)


def _gemm_body(map_ref, xs_ref, g_ref, u_ref, d_ref, ys_ref):
    b = pl.program_id(0)

    @pl.when(map_ref[b] < E)
    def _():
        xb = xs_ref[...].astype(jnp.bfloat16)
        g = jnp.dot(xb, g_ref[0].astype(jnp.bfloat16),
                    preferred_element_type=jnp.float32)
        u = jnp.dot(xb, u_ref[0].astype(jnp.bfloat16),
                    preferred_element_type=jnp.float32)
        h = (g / (1.0 + jnp.exp(-g))) * u
        ys_ref[...] = jnp.dot(h.astype(jnp.bfloat16), d_ref[0].astype(jnp.bfloat16),
                              preferred_element_type=jnp.float32)


def _grouped_gemm(block_map, xs, gate_w, up_w, down_w):
    grid_spec = pltpu.PrefetchScalarGridSpec(
        num_scalar_prefetch=1,
        grid=(NB,),
        in_specs=[
            pl.BlockSpec((BLOCK, D), lambda b, m: (jnp.where(m[b] < E, b, NB), 0)),
            pl.BlockSpec((1, D, F), lambda b, m: (jnp.minimum(m[b], E - 1), 0, 0)),
            pl.BlockSpec((1, D, F), lambda b, m: (jnp.minimum(m[b], E - 1), 0, 0)),
            pl.BlockSpec((1, F, D), lambda b, m: (jnp.minimum(m[b], E - 1), 0, 0)),
        ],
        out_specs=pl.BlockSpec((BLOCK, D), lambda b, m: (jnp.where(m[b] < E, b, NB), 0)),
    )
    return pl.pallas_call(
        _gemm_body,
        grid_spec=grid_spec,
        out_shape=jax.ShapeDtypeStruct((MAX_ROWS + BLOCK, D), jnp.float32),
    )(block_map, xs, gate_w, up_w, down_w)


def _combine_body(ys_hbm, pos0_hbm, pos1_hbm, w0_hbm, w1_hbm, out_hbm,
                  idx0_v, idx1_v, w0_v, w1_v, buf0_v, buf1_v, sem):
    wid = lax.axis_index("s") * 2 + lax.axis_index("c")
    base = wid * TPW
    pltpu.sync_copy(pos0_hbm.at[pl.ds(base, TPW)], idx0_v)
    pltpu.sync_copy(pos1_hbm.at[pl.ds(base, TPW)], idx1_v)
    pltpu.sync_copy(w0_hbm.at[pl.ds(base, TPW)], w0_v)
    pltpu.sync_copy(w1_hbm.at[pl.ds(base, TPW)], w1_v)
    c0 = pltpu.async_copy(ys_hbm.at[idx0_v], buf0_v, sem)
    c1 = pltpu.async_copy(ys_hbm.at[idx1_v], buf1_v, sem)
    c0.wait()
    c1.wait()

    def row(i, carry):
        a0 = w0_v[i, :]
        a1 = w1_v[i, :]
        for ch in range(D // L):
            sl = pl.ds(ch * L, L)
            buf0_v[i, sl] = a0 * buf0_v[i, sl] + a1 * buf1_v[i, sl]
        return carry

    lax.fori_loop(0, TPW, row, 0)
    pltpu.sync_copy(buf0_v, out_hbm.at[pl.ds(base, TPW)])


@functools.cache
def _make_combine():
    return pl.kernel(
        _combine_body,
        mesh=plsc.VectorSubcoreMesh(core_axis_name="c", subcore_axis_name="s"),
        out_type=jax.ShapeDtypeStruct((T, D), jnp.float32),
        scratch_types=[
            pltpu.VMEM((TPW,), jnp.int32),
            pltpu.VMEM((TPW,), jnp.int32),
            pltpu.VMEM((TPW, L), jnp.float32),
            pltpu.VMEM((TPW, L), jnp.float32),
            pltpu.VMEM((TPW, D), jnp.float32),
            pltpu.VMEM((TPW, D), jnp.float32),
            pltpu.SemaphoreType.DMA,
        ],
    )


def _combine(ys, pos0, pos1, w0, w1):
    return _make_combine()(ys, pos0, pos1, w0, w1)


def kernel(x, W_router, gate_w, up_w, down_w):
    pos0, pos1, w0, w1, block_map = _router(x, W_router)
    xs = _dispatch(x, pos0, pos1)
    ys = _grouped_gemm(block_map, xs, gate_w, up_w, down_w)
    return _combine(ys, pos0, pos1, w0, w1)
